# baseline (device time: 666380 ns/iter reference)
import jax
import jax.numpy as jnp
from jax import lax
from jax.experimental import pallas as pl
from jax.experimental.pallas import tpu as pltpu

N_DEV = 4
M_PER = 1024
K_PER = 1024
N_TOT = 8192
NT = 2048
N_TILES = N_TOT // NT


def kernel(x, w_mat, scale_x, scale_w):
    x8 = x.astype(jnp.float8_e5m2)
    w8 = w_mat.astype(jnp.float8_e5m2)

    def body(x_ref, w_ref, sx_ref, sw_ref, out_ref, comm_ref, send_sems, recv_sems):
        p = lax.axis_index("i")
        right = lax.rem(p + 1, N_DEV)
        left = lax.rem(p + N_DEV - 1, N_DEV)

        barrier_sem = pltpu.get_barrier_semaphore()
        for nbr in (left, right):
            pl.semaphore_signal(
                barrier_sem, inc=1,
                device_id=(nbr,), device_id_type=pl.DeviceIdType.MESH,
            )
        pl.semaphore_wait(barrier_sem, 2)

        scale = sx_ref[0] * sw_ref[0]

        def partial(c, t):
            a = x_ref[pl.ds(c * M_PER, M_PER), :]
            b = w_ref[:, t * NT:(t + 1) * NT]
            return lax.dot_general(
                a, b, (((1,), (0,)), ((), ())),
                preferred_element_type=jnp.float32,
            )

        for t in range(N_TILES):
            c0 = lax.rem(p + N_DEV - 1, N_DEV)
            comm_ref[0] = partial(c0, t).astype(jnp.bfloat16)

            for s in range(N_DEV - 1):
                send_slot = s % 2
                recv_slot = (s + 1) % 2
                rdma = pltpu.make_async_remote_copy(
                    src_ref=comm_ref.at[send_slot],
                    dst_ref=comm_ref.at[recv_slot],
                    send_sem=send_sems.at[send_slot],
                    recv_sem=recv_sems.at[recv_slot],
                    device_id=(right,),
                    device_id_type=pl.DeviceIdType.MESH,
                )
                rdma.start()
                rdma.wait()

                c = lax.rem(p + N_DEV - 2 - s + N_DEV, N_DEV)
                acc = comm_ref[recv_slot].astype(jnp.float32) + partial(c, t)
                if s < N_DEV - 2:
                    comm_ref[recv_slot] = acc.astype(jnp.bfloat16)
                else:
                    out_ref[:, t * NT:(t + 1) * NT] = jnp.maximum(
                        acc * scale, 0.0
                    )

    return pl.pallas_call(
        body,
        out_shape=jax.ShapeDtypeStruct((M_PER, N_TOT), jnp.float32),
        in_specs=[
            pl.BlockSpec(memory_space=pltpu.VMEM),
            pl.BlockSpec(memory_space=pltpu.VMEM),
            pl.BlockSpec(memory_space=pltpu.SMEM),
            pl.BlockSpec(memory_space=pltpu.SMEM),
        ],
        out_specs=pl.BlockSpec(memory_space=pltpu.VMEM),
        scratch_shapes=[
            pltpu.VMEM((2, M_PER, NT), jnp.bfloat16),
            pltpu.SemaphoreType.DMA((2,)),
            pltpu.SemaphoreType.DMA((2,)),
        ],
        compiler_params=pltpu.CompilerParams(
            collective_id=0,
            vmem_limit_bytes=64 * 1024 * 1024,
        ),
    )(x8, w8, scale_x, scale_w)


# device time: 396587 ns/iter; 1.6803x vs baseline; 1.6803x over previous
import jax
import jax.numpy as jnp
from jax import lax
from jax.experimental import pallas as pl
from jax.experimental.pallas import tpu as pltpu

N_DEV = 4
M_PER = 1024
N_TOT = 8192
HALF = N_TOT // 2
NT = 1024
N_TILES = HALF // NT


def kernel(x, w_mat, scale_x, scale_w):
    x8 = x.astype(jnp.float8_e5m2)
    w8 = w_mat.astype(jnp.float8_e5m2)

    def body(x_ref, w_ref, sx_ref, sw_ref, out_ref,
             comm_cw, comm_ccw, send_cw, recv_cw, send_ccw, recv_ccw):
        p = lax.axis_index("i")
        right = lax.rem(p + 1, N_DEV)
        left = lax.rem(p + N_DEV - 1, N_DEV)

        barrier_sem = pltpu.get_barrier_semaphore()
        for nbr in (left, right):
            pl.semaphore_signal(
                barrier_sem, inc=1,
                device_id=(nbr,), device_id_type=pl.DeviceIdType.MESH,
            )
        pl.semaphore_wait(barrier_sem, 2)

        scale = sx_ref[0] * sw_ref[0]

        def partial(c, col):
            a = x_ref[pl.ds(c * M_PER, M_PER), :]
            b = w_ref[:, col:col + NT]
            return lax.dot_general(
                a, b, (((1,), (0,)), ((), ())),
                preferred_element_type=jnp.float32,
            )

        for t in range(N_TILES):
            col_cw = t * NT
            col_ccw = HALF + t * NT
            comm_cw[0] = partial(lax.rem(p + N_DEV - 1, N_DEV), col_cw).astype(
                jnp.bfloat16)
            comm_ccw[0] = partial(lax.rem(p + 1, N_DEV), col_ccw).astype(
                jnp.bfloat16)

            for s in range(N_DEV - 1):
                ss = s % 2
                rs = (s + 1) % 2
                rdma_cw = pltpu.make_async_remote_copy(
                    src_ref=comm_cw.at[ss], dst_ref=comm_cw.at[rs],
                    send_sem=send_cw.at[ss], recv_sem=recv_cw.at[rs],
                    device_id=(right,), device_id_type=pl.DeviceIdType.MESH,
                )
                rdma_ccw = pltpu.make_async_remote_copy(
                    src_ref=comm_ccw.at[ss], dst_ref=comm_ccw.at[rs],
                    send_sem=send_ccw.at[ss], recv_sem=recv_ccw.at[rs],
                    device_id=(left,), device_id_type=pl.DeviceIdType.MESH,
                )
                rdma_cw.start()
                rdma_ccw.start()
                rdma_cw.wait()
                rdma_ccw.wait()

                c_cw = lax.rem(p + 2 * N_DEV - 2 - s, N_DEV)
                c_ccw = lax.rem(p + 2 + s, N_DEV)
                acc_cw = comm_cw[rs].astype(jnp.float32) + partial(c_cw, col_cw)
                acc_ccw = comm_ccw[rs].astype(jnp.float32) + partial(c_ccw, col_ccw)
                if s < N_DEV - 2:
                    comm_cw[rs] = acc_cw.astype(jnp.bfloat16)
                    comm_ccw[rs] = acc_ccw.astype(jnp.bfloat16)
                else:
                    out_ref[:, col_cw:col_cw + NT] = jnp.maximum(
                        acc_cw * scale, 0.0)
                    out_ref[:, col_ccw:col_ccw + NT] = jnp.maximum(
                        acc_ccw * scale, 0.0)

    return pl.pallas_call(
        body,
        out_shape=jax.ShapeDtypeStruct((M_PER, N_TOT), jnp.float32),
        in_specs=[
            pl.BlockSpec(memory_space=pltpu.VMEM),
            pl.BlockSpec(memory_space=pltpu.VMEM),
            pl.BlockSpec(memory_space=pltpu.SMEM),
            pl.BlockSpec(memory_space=pltpu.SMEM),
        ],
        out_specs=pl.BlockSpec(memory_space=pltpu.VMEM),
        scratch_shapes=[
            pltpu.VMEM((2, M_PER, NT), jnp.bfloat16),
            pltpu.VMEM((2, M_PER, NT), jnp.bfloat16),
            pltpu.SemaphoreType.DMA((2,)),
            pltpu.SemaphoreType.DMA((2,)),
            pltpu.SemaphoreType.DMA((2,)),
            pltpu.SemaphoreType.DMA((2,)),
        ],
        compiler_params=pltpu.CompilerParams(
            collective_id=0,
            vmem_limit_bytes=64 * 1024 * 1024,
        ),
    )(x8, w8, scale_x, scale_w)


# device time: 346750 ns/iter; 1.9218x vs baseline; 1.1437x over previous
import jax
import jax.numpy as jnp
from jax import lax
from jax.experimental import pallas as pl
from jax.experimental.pallas import tpu as pltpu

N_DEV = 4
M_PER = 1024
N_TOT = 8192
HALF = N_TOT // 2
NT = 1024
N_TILES = HALF // NT


def kernel(x, w_mat, scale_x, scale_w):
    x8 = x.astype(jnp.float8_e5m2)
    w8 = w_mat.astype(jnp.float8_e5m2)

    def body(x_ref, w_ref, sx_ref, sw_ref, out_ref,
             comm_cw, comm_ccw,
             send_cw, recv_cw, send_ccw, recv_ccw, credits):
        p = lax.axis_index("i")
        right = lax.rem(p + 1, N_DEV)
        left = lax.rem(p + N_DEV - 1, N_DEV)

        barrier_sem = pltpu.get_barrier_semaphore()
        for nbr in (left, right):
            pl.semaphore_signal(
                barrier_sem, inc=1,
                device_id=(nbr,), device_id_type=pl.DeviceIdType.MESH,
            )
        pl.semaphore_wait(barrier_sem, 2)

        scale = sx_ref[0] * sw_ref[0]

        def partial(c, col):
            a = x_ref[pl.ds(c * M_PER, M_PER), :]
            b = w_ref[:, col:col + NT]
            return lax.dot_general(
                a, b, (((1,), (0,)), ((), ())),
                preferred_element_type=jnp.float32,
            )

        dirs = [
            dict(di=0, comm=comm_cw, ssem=send_cw, rsem=recv_cw, peer=right,
                 up=left, base=0,
                 chunk=lambda s: lax.rem(p + 2 * N_DEV - 1 - s, N_DEV)),
            dict(di=1, comm=comm_ccw, ssem=send_ccw, rsem=recv_ccw, peer=left,
                 up=right, base=HALF,
                 chunk=lambda s: lax.rem(p + 1 + s, N_DEV)),
        ]

        def col(d, t):
            return d["base"] + t * NT

        def desc(d, t, s):
            return pltpu.make_async_remote_copy(
                src_ref=d["comm"].at[t % 2, s % 2],
                dst_ref=d["comm"].at[t % 2, (s + 1) % 2],
                send_sem=d["ssem"].at[t % 2, s],
                recv_sem=d["rsem"].at[t % 2, s],
                device_id=(d["peer"],),
                device_id_type=pl.DeviceIdType.MESH,
            )

        def p_and_send(d, t):
            d["comm"][t % 2, 0] = partial(
                d["chunk"](0), col(d, t)).astype(jnp.bfloat16)
            desc(d, t, 0).start()

        def hop(d, t, s):
            if s == 2:
                desc(d, t, 0).wait_send()
            elif t >= 2:
                desc(d, t - 2, 1).wait_send()
            desc(d, t, s - 1).wait_recv()
            acc = (d["comm"][t % 2, s % 2].astype(jnp.float32)
                   + partial(d["chunk"](s), col(d, t)))
            d["comm"][t % 2, s % 2] = acc.astype(jnp.bfloat16)
            desc(d, t, s).start()

        def epilogue(d, t, credit=False):
            desc(d, t, N_DEV - 2).wait_recv()
            acc = (d["comm"][t % 2, (N_DEV - 1) % 2].astype(jnp.float32)
                   + partial(d["chunk"](N_DEV - 1), col(d, t)))
            c = col(d, t)
            out_ref[:, c:c + NT] = jnp.maximum(acc * scale, 0.0)
            if credit:
                pl.semaphore_signal(
                    credits.at[d["di"]], inc=1,
                    device_id=(d["up"],), device_id_type=pl.DeviceIdType.MESH,
                )

        for t in (0, 1):
            for d in dirs:
                p_and_send(d, t)
        for s in (1, 2):
            for t in (0, 1):
                for d in dirs:
                    hop(d, t, s)
        for t in (0, 1):
            for d in dirs:
                epilogue(d, t, credit=True)
        for t in (2, 3):
            for d in dirs:
                desc(d, t - 2, 2).wait_send()
                pl.semaphore_wait(credits.at[d["di"]], 1)
                p_and_send(d, t)
        for s in (1, 2):
            for t in (2, 3):
                for d in dirs:
                    hop(d, t, s)
        for t in (2, 3):
            for d in dirs:
                epilogue(d, t)
        for t in (2, 3):
            for d in dirs:
                desc(d, t, 1).wait_send()
                desc(d, t, 2).wait_send()

    return pl.pallas_call(
        body,
        out_shape=jax.ShapeDtypeStruct((M_PER, N_TOT), jnp.float32),
        in_specs=[
            pl.BlockSpec(memory_space=pltpu.VMEM),
            pl.BlockSpec(memory_space=pltpu.VMEM),
            pl.BlockSpec(memory_space=pltpu.SMEM),
            pl.BlockSpec(memory_space=pltpu.SMEM),
        ],
        out_specs=pl.BlockSpec(memory_space=pltpu.MemorySpace.VMEM),
        scratch_shapes=[
            pltpu.VMEM((2, 2, M_PER, NT), jnp.bfloat16),
            pltpu.VMEM((2, 2, M_PER, NT), jnp.bfloat16),
            pltpu.SemaphoreType.DMA((2, N_DEV - 1)),
            pltpu.SemaphoreType.DMA((2, N_DEV - 1)),
            pltpu.SemaphoreType.DMA((2, N_DEV - 1)),
            pltpu.SemaphoreType.DMA((2, N_DEV - 1)),
            pltpu.SemaphoreType.REGULAR((2,)),
        ],
        compiler_params=pltpu.CompilerParams(
            collective_id=0,
            vmem_limit_bytes=64 * 1024 * 1024,
        ),
    )(x8, w8, scale_x, scale_w)


# device time: 338155 ns/iter; 1.9706x vs baseline; 1.0254x over previous
import jax
import jax.numpy as jnp
from jax import lax
from jax.experimental import pallas as pl
from jax.experimental.pallas import tpu as pltpu

N_DEV = 4
M_PER = 1024
N_TOT = 8192
HALF = N_TOT // 2
NT = 1024
N_TILES = HALF // NT


def kernel(x, w_mat, scale_x, scale_w):
    x8 = x.astype(jnp.float8_e5m2)
    w8 = w_mat.astype(jnp.float8_e5m2)

    def body(x_ref, w_ref, sx_ref, sw_ref, out_ref,
             comm_cw, comm_ccw, stage,
             send_cw, recv_cw, send_ccw, recv_ccw, credits, stage_sems):
        p = lax.axis_index("i")
        right = lax.rem(p + 1, N_DEV)
        left = lax.rem(p + N_DEV - 1, N_DEV)

        barrier_sem = pltpu.get_barrier_semaphore()
        for nbr in (left, right):
            pl.semaphore_signal(
                barrier_sem, inc=1,
                device_id=(nbr,), device_id_type=pl.DeviceIdType.MESH,
            )
        pl.semaphore_wait(barrier_sem, 2)

        scale = sx_ref[0] * sw_ref[0]

        def partial(c, col):
            a = x_ref[pl.ds(c * M_PER, M_PER), :]
            b = w_ref[:, col:col + NT]
            return lax.dot_general(
                a, b, (((1,), (0,)), ((), ())),
                preferred_element_type=jnp.float32,
            )

        dirs = [
            dict(di=0, comm=comm_cw, ssem=send_cw, rsem=recv_cw, peer=right,
                 up=left, base=0,
                 chunk=lambda s: lax.rem(p + 2 * N_DEV - 1 - s, N_DEV)),
            dict(di=1, comm=comm_ccw, ssem=send_ccw, rsem=recv_ccw, peer=left,
                 up=right, base=HALF,
                 chunk=lambda s: lax.rem(p + 1 + s, N_DEV)),
        ]

        def col(d, t):
            return d["base"] + t * NT

        def desc(d, t, s):
            return pltpu.make_async_remote_copy(
                src_ref=d["comm"].at[t % 2, s % 2],
                dst_ref=d["comm"].at[t % 2, (s + 1) % 2],
                send_sem=d["ssem"].at[t % 2, s],
                recv_sem=d["rsem"].at[t % 2, s],
                device_id=(d["peer"],),
                device_id_type=pl.DeviceIdType.MESH,
            )

        def stage_copy(d, t):
            return pltpu.make_async_copy(
                stage.at[d["di"], t % 2],
                out_ref.at[:, pl.ds(col(d, t), NT)],
                stage_sems.at[d["di"], t % 2],
            )

        def p_and_send(d, t):
            d["comm"][t % 2, 0] = partial(
                d["chunk"](0), col(d, t)).astype(jnp.bfloat16)
            desc(d, t, 0).start()

        def hop(d, t, s):
            if s == 2:
                desc(d, t, 0).wait_send()
            elif t >= 2:
                desc(d, t - 2, 1).wait_send()
            desc(d, t, s - 1).wait_recv()
            acc = (d["comm"][t % 2, s % 2].astype(jnp.float32)
                   + partial(d["chunk"](s), col(d, t)))
            d["comm"][t % 2, s % 2] = acc.astype(jnp.bfloat16)
            desc(d, t, s).start()

        def epilogue(d, t, credit=False):
            desc(d, t, N_DEV - 2).wait_recv()
            acc = (d["comm"][t % 2, (N_DEV - 1) % 2].astype(jnp.float32)
                   + partial(d["chunk"](N_DEV - 1), col(d, t)))
            if t >= 2:
                stage_copy(d, t - 2).wait()
            stage[d["di"], t % 2] = jnp.maximum(acc * scale, 0.0)
            stage_copy(d, t).start()
            if credit:
                pl.semaphore_signal(
                    credits.at[d["di"]], inc=1,
                    device_id=(d["up"],), device_id_type=pl.DeviceIdType.MESH,
                )

        for t in (0, 1):
            for d in dirs:
                p_and_send(d, t)
        for s in (1, 2):
            for t in (0, 1):
                for d in dirs:
                    hop(d, t, s)
        for t in (0, 1):
            for d in dirs:
                epilogue(d, t, credit=True)
        for t in (2, 3):
            for d in dirs:
                desc(d, t - 2, 2).wait_send()
                pl.semaphore_wait(credits.at[d["di"]], 1)
                p_and_send(d, t)
        for s in (1, 2):
            for t in (2, 3):
                for d in dirs:
                    hop(d, t, s)
        for t in (2, 3):
            for d in dirs:
                epilogue(d, t)
        for t in (2, 3):
            for d in dirs:
                desc(d, t, 1).wait_send()
                desc(d, t, 2).wait_send()
                stage_copy(d, t).wait()

    return pl.pallas_call(
        body,
        out_shape=jax.ShapeDtypeStruct((M_PER, N_TOT), jnp.float32),
        in_specs=[
            pl.BlockSpec(memory_space=pltpu.VMEM),
            pl.BlockSpec(memory_space=pltpu.VMEM),
            pl.BlockSpec(memory_space=pltpu.SMEM),
            pl.BlockSpec(memory_space=pltpu.SMEM),
        ],
        out_specs=pl.BlockSpec(memory_space=pl.ANY),
        scratch_shapes=[
            pltpu.VMEM((2, 2, M_PER, NT), jnp.bfloat16),
            pltpu.VMEM((2, 2, M_PER, NT), jnp.bfloat16),
            pltpu.VMEM((2, 2, M_PER, NT), jnp.float32),
            pltpu.SemaphoreType.DMA((2, N_DEV - 1)),
            pltpu.SemaphoreType.DMA((2, N_DEV - 1)),
            pltpu.SemaphoreType.DMA((2, N_DEV - 1)),
            pltpu.SemaphoreType.DMA((2, N_DEV - 1)),
            pltpu.SemaphoreType.REGULAR((2,)),
            pltpu.SemaphoreType.DMA((2, 2)),
        ],
        compiler_params=pltpu.CompilerParams(
            collective_id=0,
            vmem_limit_bytes=64 * 1024 * 1024,
        ),
    )(x8, w8, scale_x, scale_w)


# device time: 316363 ns/iter; 2.1064x vs baseline; 1.0689x over previous
import jax
import jax.numpy as jnp
from jax import lax
from jax.experimental import pallas as pl
from jax.experimental.pallas import tpu as pltpu

N_DEV = 4
M_PER = 1024
M_TOT = 4096
K_PER = 1024
N_TOT = 8192
HALF = N_TOT // 2
NT = 1024
N_TILES = HALF // NT

W_ORDER = [0, 4, 1, 5, 2, 6, 3, 7]


def kernel(x, w_mat, scale_x, scale_w):
    def body(x_hbm, w_hbm, sx_ref, sw_ref, out_ref,
             x8, w8, xf, wf, comm_cw, comm_ccw, stage,
             xdma_sems, wdma_sems,
             send_cw, recv_cw, send_ccw, recv_ccw, credits, stage_sems):
        p = lax.axis_index("i")
        right = lax.rem(p + 1, N_DEV)
        left = lax.rem(p + N_DEV - 1, N_DEV)

        x_order = [lax.rem(p + N_DEV - 1, N_DEV), lax.rem(p + 1, N_DEV),
                   lax.rem(p + 2, N_DEV), p]

        def xdma(i):
            c = x_order[i]
            return pltpu.make_async_copy(
                x_hbm.at[pl.ds(c * M_PER, M_PER), :],
                xf.at[i % 2], xdma_sems.at[i % 2])

        def xcast(i):
            c = x_order[i]
            x8[pl.ds(c * M_PER, M_PER), :] = xf[i % 2].astype(jnp.float8_e5m2)

        def wdma(k):
            j = W_ORDER[k]
            return pltpu.make_async_copy(
                w_hbm.at[:, pl.ds(j * NT, NT)],
                wf.at[k % 2], wdma_sems.at[k % 2])

        def wcast(k):
            j = W_ORDER[k]
            w8[:, j * NT:(j + 1) * NT] = wf[k % 2].astype(jnp.float8_e5m2)

        xdma(0).start()
        xdma(1).start()
        wdma(0).start()
        wdma(1).start()

        barrier_sem = pltpu.get_barrier_semaphore()
        for nbr in (left, right):
            pl.semaphore_signal(
                barrier_sem, inc=1,
                device_id=(nbr,), device_id_type=pl.DeviceIdType.MESH,
            )
        pl.semaphore_wait(barrier_sem, 2)

        scale = sx_ref[0] * sw_ref[0]

        def partial(c, col):
            a = x8[pl.ds(c * M_PER, M_PER), :]
            b = w8[:, col:col + NT]
            return lax.dot_general(
                a, b, (((1,), (0,)), ((), ())),
                preferred_element_type=jnp.float32,
            )

        dirs = [
            dict(di=0, comm=comm_cw, ssem=send_cw, rsem=recv_cw, peer=right,
                 up=left, base=0,
                 chunk=lambda s: lax.rem(p + 2 * N_DEV - 1 - s, N_DEV)),
            dict(di=1, comm=comm_ccw, ssem=send_ccw, rsem=recv_ccw, peer=left,
                 up=right, base=HALF,
                 chunk=lambda s: lax.rem(p + 1 + s, N_DEV)),
        ]

        def col(d, t):
            return d["base"] + t * NT

        def desc(d, t, s):
            return pltpu.make_async_remote_copy(
                src_ref=d["comm"].at[t % 2, s % 2],
                dst_ref=d["comm"].at[t % 2, (s + 1) % 2],
                send_sem=d["ssem"].at[t % 2, s],
                recv_sem=d["rsem"].at[t % 2, s],
                device_id=(d["peer"],),
                device_id_type=pl.DeviceIdType.MESH,
            )

        def stage_copy(d, t):
            return pltpu.make_async_copy(
                stage.at[d["di"], t % 2],
                out_ref.at[:, pl.ds(col(d, t), NT)],
                stage_sems.at[d["di"], t % 2],
            )

        def p_and_send(d, t):
            d["comm"][t % 2, 0] = partial(
                d["chunk"](0), col(d, t)).astype(jnp.bfloat16)
            desc(d, t, 0).start()

        def hop(d, t, s):
            if s == 2:
                desc(d, t, 0).wait_send()
            elif t >= 2:
                desc(d, t - 2, 1).wait_send()
            desc(d, t, s - 1).wait_recv()
            acc = (d["comm"][t % 2, s % 2].astype(jnp.float32)
                   + partial(d["chunk"](s), col(d, t)))
            d["comm"][t % 2, s % 2] = acc.astype(jnp.bfloat16)
            desc(d, t, s).start()

        def epilogue(d, t, credit=False):
            desc(d, t, N_DEV - 2).wait_recv()
            acc = (d["comm"][t % 2, (N_DEV - 1) % 2].astype(jnp.float32)
                   + partial(d["chunk"](N_DEV - 1), col(d, t)))
            if t >= 2:
                stage_copy(d, t - 2).wait()
            stage[d["di"], t % 2] = jnp.maximum(acc * scale, 0.0)
            stage_copy(d, t).start()
            if credit:
                pl.semaphore_signal(
                    credits.at[d["di"]], inc=1,
                    device_id=(d["up"],), device_id_type=pl.DeviceIdType.MESH,
                )

        for i in (0, 1):
            xdma(i).wait()
            xcast(i)
            xdma(i + 2).start()
        for k in (0, 1):
            wdma(k).wait()
            wcast(k)
            wdma(k + 2).start()
        for d in dirs:
            p_and_send(d, 0)
        for k in (2, 3):
            wdma(k).wait()
            wcast(k)
            wdma(k + 2).start()
        for d in dirs:
            p_and_send(d, 1)
        for i in (2, 3):
            xdma(i).wait()
            xcast(i)
        for k in (4, 5):
            wdma(k).wait()
            wcast(k)
            wdma(k + 2).start()
        for s in (1, 2):
            for t in (0, 1):
                for d in dirs:
                    hop(d, t, s)
        for k in (6, 7):
            wdma(k).wait()
            wcast(k)
        for t in (0, 1):
            for d in dirs:
                epilogue(d, t, credit=True)
        for t in (2, 3):
            for d in dirs:
                desc(d, t - 2, 2).wait_send()
                pl.semaphore_wait(credits.at[d["di"]], 1)
                p_and_send(d, t)
        for s in (1, 2):
            for t in (2, 3):
                for d in dirs:
                    hop(d, t, s)
        for t in (2, 3):
            for d in dirs:
                epilogue(d, t)
        for t in (2, 3):
            for d in dirs:
                desc(d, t, 1).wait_send()
                desc(d, t, 2).wait_send()
                stage_copy(d, t).wait()

    return pl.pallas_call(
        body,
        out_shape=jax.ShapeDtypeStruct((M_PER, N_TOT), jnp.float32),
        in_specs=[
            pl.BlockSpec(memory_space=pl.ANY),
            pl.BlockSpec(memory_space=pl.ANY),
            pl.BlockSpec(memory_space=pltpu.SMEM),
            pl.BlockSpec(memory_space=pltpu.SMEM),
        ],
        out_specs=pl.BlockSpec(memory_space=pl.ANY),
        scratch_shapes=[
            pltpu.VMEM((M_TOT, K_PER), jnp.float8_e5m2),
            pltpu.VMEM((K_PER, N_TOT), jnp.float8_e5m2),
            pltpu.VMEM((2, M_PER, K_PER), jnp.float32),
            pltpu.VMEM((2, K_PER, NT), jnp.float32),
            pltpu.VMEM((2, 2, M_PER, NT), jnp.bfloat16),
            pltpu.VMEM((2, 2, M_PER, NT), jnp.bfloat16),
            pltpu.VMEM((2, 2, M_PER, NT), jnp.float32),
            pltpu.SemaphoreType.DMA((2,)),
            pltpu.SemaphoreType.DMA((2,)),
            pltpu.SemaphoreType.DMA((2, N_DEV - 1)),
            pltpu.SemaphoreType.DMA((2, N_DEV - 1)),
            pltpu.SemaphoreType.DMA((2, N_DEV - 1)),
            pltpu.SemaphoreType.DMA((2, N_DEV - 1)),
            pltpu.SemaphoreType.REGULAR((2,)),
            pltpu.SemaphoreType.DMA((2, 2)),
        ],
        compiler_params=pltpu.CompilerParams(
            collective_id=0,
            vmem_limit_bytes=64 * 1024 * 1024,
        ),
    )(x, w_mat, scale_x, scale_w)


# device time: 271063 ns/iter; 2.4584x vs baseline; 1.1671x over previous
import jax
import jax.numpy as jnp
from jax import lax
from jax.experimental import pallas as pl
from jax.experimental.pallas import tpu as pltpu

N_DEV = 4
M_PER = 1024
M_TOT = 4096
K_PER = 1024
N_TOT = 8192
HALF = N_TOT // 2
NT = 1024
N_TILES = HALF // NT

W_ORDER = [0, 4, 1, 5, 2, 6, 3, 7]


def kernel(x, w_mat, scale_x, scale_w):
    def body(x_hbm, w_hbm, sx_ref, sw_ref, out_ref,
             x8, w8, comm_cw, comm_ccw, stage,
             inj_src_cw, inj_dst_cw, inj_src_ccw, inj_dst_ccw,
             xdma_sems, wdma_sems,
             send_cw, recv_cw, send_ccw, recv_ccw, credits, stage_sems):
        p = lax.axis_index("i")
        right = lax.rem(p + 1, N_DEV)
        left = lax.rem(p + N_DEV - 1, N_DEV)

        x_order = [lax.rem(p + N_DEV - 1, N_DEV), lax.rem(p + 1, N_DEV),
                   lax.rem(p + 2, N_DEV), p]

        def xdma(i):
            c = x_order[i]
            return pltpu.make_async_copy(
                x_hbm.at[pl.ds(c * M_PER, M_PER), :],
                stage.at[0, i % 2], xdma_sems.at[i % 2])

        def xcast(i):
            c = x_order[i]
            x8[pl.ds(c * M_PER, M_PER), :] = stage[0, i % 2].astype(
                jnp.float8_e5m2)

        def wdma(k):
            j = W_ORDER[k]
            return pltpu.make_async_copy(
                w_hbm.at[:, pl.ds(j * NT, NT)],
                stage.at[1, k % 2], wdma_sems.at[k % 2])

        def wcast(k):
            j = W_ORDER[k]
            w8[:, j * NT:(j + 1) * NT] = stage[1, k % 2].astype(
                jnp.float8_e5m2)

        xdma(0).start()
        xdma(1).start()
        wdma(0).start()
        wdma(1).start()

        barrier_sem = pltpu.get_barrier_semaphore()
        for nbr in (left, right):
            pl.semaphore_signal(
                barrier_sem, inc=1,
                device_id=(nbr,), device_id_type=pl.DeviceIdType.MESH,
            )
        pl.semaphore_wait(barrier_sem, 2)

        scale = sx_ref[0] * sw_ref[0]

        def partial(c, col):
            a = x8[pl.ds(c * M_PER, M_PER), :]
            b = w8[:, col:col + NT]
            return lax.dot_general(
                a, b, (((1,), (0,)), ((), ())),
                preferred_element_type=jnp.float32,
            )

        dirs = [
            dict(di=0, comm=comm_cw, ssem=send_cw, rsem=recv_cw, peer=right,
                 up=left, base=0, isrc=inj_src_cw, idst=inj_dst_cw,
                 chunk=lambda s: lax.rem(p + 2 * N_DEV - 1 - s, N_DEV)),
            dict(di=1, comm=comm_ccw, ssem=send_ccw, rsem=recv_ccw, peer=left,
                 up=right, base=HALF, isrc=inj_src_ccw, idst=inj_dst_ccw,
                 chunk=lambda s: lax.rem(p + 1 + s, N_DEV)),
        ]

        def col(d, t):
            return d["base"] + t * NT

        def desc(d, t, s):
            if s == 0:
                src, dst = d["isrc"].at[t % 2], d["idst"].at[t % 2]
            else:
                src = d["comm"].at[t % 2, s % 2]
                dst = d["comm"].at[t % 2, (s + 1) % 2]
            return pltpu.make_async_remote_copy(
                src_ref=src, dst_ref=dst,
                send_sem=d["ssem"].at[t % 2, s],
                recv_sem=d["rsem"].at[t % 2, s],
                device_id=(d["peer"],),
                device_id_type=pl.DeviceIdType.MESH,
            )

        def stage_copy(d, t):
            return pltpu.make_async_copy(
                stage.at[d["di"], t % 2],
                out_ref.at[:, pl.ds(col(d, t), NT)],
                stage_sems.at[d["di"], t % 2],
            )

        def p_and_send(d, t):
            d["isrc"][t % 2] = partial(
                d["chunk"](0), col(d, t)).astype(jnp.float8_e4m3fn)
            desc(d, t, 0).start()

        def hop(d, t, s):
            if s == 2:
                desc(d, t, 0).wait_send()
            elif t >= 2:
                desc(d, t - 2, 1).wait_send()
            desc(d, t, s - 1).wait_recv()
            recv = d["idst"][t % 2] if s == 1 else d["comm"][t % 2, s % 2]
            acc = recv.astype(jnp.float32) + partial(d["chunk"](s), col(d, t))
            d["comm"][t % 2, s % 2] = acc.astype(jnp.bfloat16)
            desc(d, t, s).start()

        def epilogue(d, t, credit=False):
            desc(d, t, N_DEV - 2).wait_recv()
            acc = (d["comm"][t % 2, (N_DEV - 1) % 2].astype(jnp.float32)
                   + partial(d["chunk"](N_DEV - 1), col(d, t)))
            if t >= 2:
                stage_copy(d, t - 2).wait()
            stage[d["di"], t % 2] = jnp.maximum(acc * scale, 0.0)
            stage_copy(d, t).start()
            if credit:
                pl.semaphore_signal(
                    credits.at[d["di"]], inc=1,
                    device_id=(d["up"],), device_id_type=pl.DeviceIdType.MESH,
                )

        for i in (0, 1):
            xdma(i).wait()
            xcast(i)
            xdma(i + 2).start()
        for k in (0, 1):
            wdma(k).wait()
            wcast(k)
            wdma(k + 2).start()
        for d in dirs:
            p_and_send(d, 0)
        for k in (2, 3):
            wdma(k).wait()
            wcast(k)
            wdma(k + 2).start()
        for d in dirs:
            p_and_send(d, 1)
        for i in (2, 3):
            xdma(i).wait()
            xcast(i)
        for k in (4, 5):
            wdma(k).wait()
            wcast(k)
            wdma(k + 2).start()
        for s in (1, 2):
            for t in (0, 1):
                for d in dirs:
                    hop(d, t, s)
        for k in (6, 7):
            wdma(k).wait()
            wcast(k)
        for t in (0, 1):
            for d in dirs:
                epilogue(d, t, credit=True)
        for t in (2, 3):
            for d in dirs:
                desc(d, t - 2, 2).wait_send()
                pl.semaphore_wait(credits.at[d["di"]], 1)
                p_and_send(d, t)
        for s in (1, 2):
            for t in (2, 3):
                for d in dirs:
                    hop(d, t, s)
        for t in (2, 3):
            for d in dirs:
                epilogue(d, t)
        for t in (2, 3):
            for d in dirs:
                desc(d, t, 1).wait_send()
                desc(d, t, 2).wait_send()
                stage_copy(d, t).wait()

    return pl.pallas_call(
        body,
        out_shape=jax.ShapeDtypeStruct((M_PER, N_TOT), jnp.float32),
        in_specs=[
            pl.BlockSpec(memory_space=pl.ANY),
            pl.BlockSpec(memory_space=pl.ANY),
            pl.BlockSpec(memory_space=pltpu.SMEM),
            pl.BlockSpec(memory_space=pltpu.SMEM),
        ],
        out_specs=pl.BlockSpec(memory_space=pl.ANY),
        scratch_shapes=[
            pltpu.VMEM((M_TOT, K_PER), jnp.float8_e5m2),
            pltpu.VMEM((K_PER, N_TOT), jnp.float8_e5m2),
            pltpu.VMEM((2, 2, M_PER, NT), jnp.bfloat16),
            pltpu.VMEM((2, 2, M_PER, NT), jnp.bfloat16),
            pltpu.VMEM((2, 2, M_PER, NT), jnp.float32),
            pltpu.VMEM((2, M_PER, NT), jnp.float8_e4m3fn),
            pltpu.VMEM((2, M_PER, NT), jnp.float8_e4m3fn),
            pltpu.VMEM((2, M_PER, NT), jnp.float8_e4m3fn),
            pltpu.VMEM((2, M_PER, NT), jnp.float8_e4m3fn),
            pltpu.SemaphoreType.DMA((2,)),
            pltpu.SemaphoreType.DMA((2,)),
            pltpu.SemaphoreType.DMA((2, N_DEV - 1)),
            pltpu.SemaphoreType.DMA((2, N_DEV - 1)),
            pltpu.SemaphoreType.DMA((2, N_DEV - 1)),
            pltpu.SemaphoreType.DMA((2, N_DEV - 1)),
            pltpu.SemaphoreType.REGULAR((2,)),
            pltpu.SemaphoreType.DMA((2, 2)),
        ],
        compiler_params=pltpu.CompilerParams(
            collective_id=0,
            vmem_limit_bytes=64 * 1024 * 1024,
        ),
    )(x, w_mat, scale_x, scale_w)
